# baseline (speedup 1.0000x reference)
import jax
import jax.numpy as jnp
from jax import lax
from jax.experimental import pallas as pl
from jax.experimental.pallas import tpu as pltpu
from jax.experimental.pallas import tpu_sc as plsc

N = 50000
E = 800000
D_IN = 128
D_H = 64
D_OUT = 2
QW = 16
W2 = 8

R = 51200
DUMMY = N
RPT = R // 16
WCH = 400

EP = 802816
EROWS = EP // 128
NCH = 49
CPT = EP // 16 // 128

BLK = 2000
NBLK = N // BLK


def _mesh():
    return plsc.VectorSubcoreMesh(core_axis_name="c", subcore_axis_name="s")


_SC_PARAMS = pltpu.CompilerParams(use_tc_tiling_on_sc=False)


def _edge_sweep(tab, edges, et, poff, spm, rows_v, idxs_v, idxd_v, sem, s):
    def ch(k, carry):
        row0 = s * CPT + k * 8
        pltpu.sync_copy(edges.at[et, 0, pl.ds(row0, 8)], idxs_v)
        pltpu.sync_copy(edges.at[et, 1, pl.ds(row0, 8)], idxd_v)
        offv = jnp.zeros((16,), jnp.int32) + poff

        def ob(r, carry2):
            for l in range(8):
                idxs_v[r, pl.ds(l * 16, 16)] = (
                    idxs_v[r, pl.ds(l * 16, 16)] + offv
                )
            return carry2

        lax.fori_loop(0, 8, ob, 0)
        descs = []
        for j in range(8):
            descs.append(
                pltpu.async_copy(
                    tab.at[idxs_v.at[j]],
                    rows_v.at[pl.ds(j * 128, 128)],
                    sem,
                )
            )
        for j in range(8):
            descs[j].wait()
            pltpu.sync_copy(
                rows_v.at[pl.ds(j * 128, 128)],
                spm.at[idxd_v.at[j]],
                add=True,
            )
        return carry

    lax.fori_loop(0, NCH, ch, 0)


def _zero_spm_rows(zbuf, spm, s):
    for w in range(RPT // WCH):
        pltpu.sync_copy(zbuf, spm.at[pl.ds(s * RPT + w * WCH, WCH)])


def _sc_degrees_body(edges, ones_hbm, zeros_hbm, deg, idx_v, ones_v, zbuf, obuf, spm):
    c = lax.axis_index("c")
    s = lax.axis_index("s")
    pltpu.sync_copy(ones_hbm, ones_v)
    pltpu.sync_copy(zeros_hbm, zbuf)

    for j in range(2):
        _zero_spm_rows(zbuf, spm, s)
        plsc.subcore_barrier()

        def hchunk(k, carry):
            row0 = s * CPT + k * 8
            pltpu.sync_copy(edges.at[c, j, pl.ds(row0, 8)], idx_v)
            for j8 in range(8):
                pltpu.sync_copy(ones_v, spm.at[idx_v.at[j8]], add=True)
            return carry

        lax.fori_loop(0, NCH, hchunk, 0)
        plsc.subcore_barrier()

        for w in range(RPT // WCH):
            r0 = s * RPT + w * WCH
            pltpu.sync_copy(spm.at[pl.ds(r0, WCH)], obuf)
            pltpu.sync_copy(obuf, deg.at[2 * c + j, pl.ds(r0, WCH)])


def _sc_degrees(edges, ones8, zeros8):
    return pl.kernel(
        _sc_degrees_body,
        out_type=jax.ShapeDtypeStruct((4, R, W2), jnp.float32),
        mesh=_mesh(),
        compiler_params=_SC_PARAMS,
        scratch_types=[
            pltpu.VMEM((8, 128), jnp.int32),
            pltpu.VMEM((128, W2), jnp.float32),
            pltpu.VMEM((WCH, W2), jnp.float32),
            pltpu.VMEM((WCH, W2), jnp.float32),
            pltpu.VMEM_SHARED((R, W2), jnp.float32),
        ],
    )(edges, ones8, zeros8)


def _sc_conv1_body(tab_a, tab_b, edges, zeros_hbm, m_out,
                   idxs_v, idxd_v, rows_v, zbuf, spm, sem):
    c = lax.axis_index("c")
    s = lax.axis_index("s")
    pltpu.sync_copy(zeros_hbm, zbuf)

    for et, tab in ((0, tab_a), (1, tab_b)):
        for p in range(2):
            q = 2 * c + p
            _zero_spm_rows(zbuf, spm, s)
            plsc.subcore_barrier()
            _edge_sweep(tab, edges, et, q * R, spm, rows_v, idxs_v, idxd_v, sem, s)
            plsc.subcore_barrier()
            for w in range(RPT // WCH):
                r0 = s * RPT + w * WCH
                pltpu.sync_copy(spm.at[pl.ds(r0, WCH)], rows_v.at[pl.ds(0, WCH)])
                pltpu.sync_copy(
                    rows_v.at[pl.ds(0, WCH)],
                    m_out.at[et, q, pl.ds(r0, WCH)],
                )
            plsc.subcore_barrier()


def _sc_conv1(tab_a, tab_b, edges, zeros16):
    return pl.kernel(
        _sc_conv1_body,
        out_type=jax.ShapeDtypeStruct((2, 4, R, QW), jnp.float32),
        mesh=_mesh(),
        compiler_params=_SC_PARAMS,
        scratch_types=[
            pltpu.VMEM((8, 128), jnp.int32),
            pltpu.VMEM((8, 128), jnp.int32),
            pltpu.VMEM((1024, QW), jnp.float32),
            pltpu.VMEM((WCH, QW), jnp.float32),
            pltpu.VMEM_SHARED((R, QW), jnp.float32),
            pltpu.SemaphoreType.DMA,
        ],
    )(tab_a, tab_b, edges, zeros16)


def _sc_conv2_body(tab, edges, zeros_hbm, o_out,
                   idxs_v, idxd_v, rows_v, zbuf, spm, sem):
    c = lax.axis_index("c")
    s = lax.axis_index("s")
    pltpu.sync_copy(zeros_hbm, zbuf)
    _zero_spm_rows(zbuf, spm, s)
    plsc.subcore_barrier()
    _edge_sweep(tab, edges, c, c * R, spm, rows_v, idxs_v, idxd_v, sem, s)
    plsc.subcore_barrier()
    for w in range(RPT // WCH):
        r0 = s * RPT + w * WCH
        pltpu.sync_copy(spm.at[pl.ds(r0, WCH)], rows_v.at[pl.ds(0, WCH)])
        pltpu.sync_copy(rows_v.at[pl.ds(0, WCH)], o_out.at[c, pl.ds(r0, WCH)])


def _sc_conv2(tab, edges, zeros8):
    return pl.kernel(
        _sc_conv2_body,
        out_type=jax.ShapeDtypeStruct((2, R, W2), jnp.float32),
        mesh=_mesh(),
        compiler_params=_SC_PARAMS,
        scratch_types=[
            pltpu.VMEM((8, 128), jnp.int32),
            pltpu.VMEM((8, 128), jnp.int32),
            pltpu.VMEM((1024, W2), jnp.float32),
            pltpu.VMEM((WCH, W2), jnp.float32),
            pltpu.VMEM_SHARED((R, W2), jnp.float32),
            pltpu.SemaphoreType.DMA,
        ],
    )(tab, edges, zeros8)


def _encoder(x, p):
    h = jnp.maximum(jnp.dot(x, p[0], preferred_element_type=jnp.float32) + p[1], 0.0)
    for j in range(4):
        h = h + jnp.maximum(
            jnp.dot(h, p[2 + 2 * j], preferred_element_type=jnp.float32)
            + p[3 + 2 * j],
            0.0,
        )
    return jnp.dot(h, p[10], preferred_element_type=jnp.float32) + p[11]


def _dense1_body(*refs):
    xu_ref, xi_ref, scl_ref = refs[0], refs[1], refs[2]
    pu = [r[...] for r in refs[3:15]]
    pi = [r[...] for r in refs[15:27]]
    w1u, w1i = refs[27][...], refs[28][...]
    ou_ref, oi_ref = refs[29], refs[30]

    scl = scl_ref[...]
    hu = _encoder(xu_ref[...], pu)
    hi = _encoder(xi_ref[...], pi)
    ou_ref[...] = jnp.dot(hu, w1u, preferred_element_type=jnp.float32) * scl[:, 0:1]
    oi_ref[...] = jnp.dot(hi, w1i, preferred_element_type=jnp.float32) * scl[:, 2:3]


def _full_spec(a):
    nd = a.ndim
    return pl.BlockSpec(a.shape, lambda i, _nd=nd: (0,) * _nd)


def _tc_dense1(x_user, x_item, scl, pu, pi, w1u, w1i):
    in_specs = [
        pl.BlockSpec((BLK, D_IN), lambda i: (i, 0)),
        pl.BlockSpec((BLK, D_IN), lambda i: (i, 0)),
        pl.BlockSpec((BLK, 4), lambda i: (i, 0)),
    ]
    args = [x_user, x_item, scl]
    for a in list(pu) + list(pi) + [w1u, w1i]:
        in_specs.append(_full_spec(a))
        args.append(a)
    out_shape = [jax.ShapeDtypeStruct((R, D_H), jnp.float32)] * 2
    out_specs = [pl.BlockSpec((BLK, D_H), lambda i: (i, 0))] * 2
    return pl.pallas_call(
        _dense1_body,
        grid=(NBLK,),
        in_specs=in_specs,
        out_specs=out_specs,
        out_shape=out_shape,
    )(*args)


def _prep2_body(m_ref, scl_ref, w2u_ref, w2i_ref, o_ref):
    mm = m_ref[...]
    scl = scl_ref[...]
    h_u1 = jnp.maximum(mm[1] * scl[:, 3:4], 0.0)
    h_i1 = jnp.maximum(mm[0] * scl[:, 1:2], 0.0)
    o_ref[0] = (
        jnp.dot(h_u1, w2u_ref[...], preferred_element_type=jnp.float32)
        * scl[:, 0:1]
    )
    o_ref[1] = (
        jnp.dot(h_i1, w2i_ref[...], preferred_element_type=jnp.float32)
        * scl[:, 2:3]
    )


def _tc_prep2(m64, scl, w2u, w2i):
    return pl.pallas_call(
        _prep2_body,
        grid=(NBLK,),
        in_specs=[
            pl.BlockSpec((2, BLK, D_H), lambda i: (0, i, 0)),
            pl.BlockSpec((BLK, 4), lambda i: (i, 0)),
            _full_spec(w2u),
            _full_spec(w2i),
        ],
        out_specs=pl.BlockSpec((2, BLK, W2), lambda i: (0, i, 0)),
        out_shape=jax.ShapeDtypeStruct((2, R, W2), jnp.float32),
    )(m64, scl, w2u, w2i)


def _scales_body(deg_ref, o_ref):
    o_ref[...] = lax.rsqrt(jnp.maximum(deg_ref[...], 1.0))


def _tc_scales(deg):
    nb = 8
    return pl.pallas_call(
        _scales_body,
        grid=(nb,),
        in_specs=[pl.BlockSpec((4, R // nb, W2), lambda i: (0, i, 0))],
        out_specs=pl.BlockSpec((4, R // nb, W2), lambda i: (0, i, 0)),
        out_shape=jax.ShapeDtypeStruct((4, R, W2), jnp.float32),
    )(deg)


def _final_body(o_ref, scl_ref, out_ref):
    oo = o_ref[...]
    scl = scl_ref[...]
    out_ref[0] = oo[0] * scl[:, 1:2]
    out_ref[1] = oo[1] * scl[:, 3:4]


def _tc_final(o_pre, scl):
    return pl.pallas_call(
        _final_body,
        grid=(NBLK,),
        in_specs=[
            pl.BlockSpec((2, BLK, W2), lambda i: (0, i, 0)),
            pl.BlockSpec((BLK, 4), lambda i: (i, 0)),
        ],
        out_specs=pl.BlockSpec((2, BLK, W2), lambda i: (0, i, 0)),
        out_shape=jax.ShapeDtypeStruct((2, R, W2), jnp.float32),
    )(o_pre, scl)


def _pad_edges(ei):
    pad = jnp.full((EP - E,), DUMMY, jnp.int32)
    return jnp.stack(
        [jnp.concatenate([ei[0], pad]), jnp.concatenate([ei[1], pad])]
    )


def kernel(x_user, x_item, ei_user_item, ei_item_user, enc_user_W_in, enc_user_b_in, enc_user_W0, enc_user_b0, enc_user_W1, enc_user_b1, enc_user_W2, enc_user_b2, enc_user_W3, enc_user_b3, enc_user_W_out, enc_user_b_out, enc_item_W_in, enc_item_b_in, enc_item_W0, enc_item_b0, enc_item_W1, enc_item_b1, enc_item_W2, enc_item_b2, enc_item_W3, enc_item_b3, enc_item_W_out, enc_item_b_out, gcn1_W_ui, gcn1_W_iu, gcn2_W_ui, gcn2_W_iu):
    edges = jnp.stack(
        [_pad_edges(ei_user_item), _pad_edges(ei_item_user)]
    ).reshape(2, 2, EROWS, 128)

    ones8 = jnp.ones((128, W2), jnp.float32)
    zeros8 = jnp.zeros((WCH, W2), jnp.float32)
    zeros16 = jnp.zeros((WCH, QW), jnp.float32)

    deg = _sc_degrees(edges, ones8, zeros8)
    scl = _tc_scales(deg)[:, :, 0].T

    pu = [enc_user_W_in, enc_user_b_in.reshape(1, D_H),
          enc_user_W0, enc_user_b0.reshape(1, D_H),
          enc_user_W1, enc_user_b1.reshape(1, D_H),
          enc_user_W2, enc_user_b2.reshape(1, D_H),
          enc_user_W3, enc_user_b3.reshape(1, D_H),
          enc_user_W_out, enc_user_b_out.reshape(1, D_H)]
    pi = [enc_item_W_in, enc_item_b_in.reshape(1, D_H),
          enc_item_W0, enc_item_b0.reshape(1, D_H),
          enc_item_W1, enc_item_b1.reshape(1, D_H),
          enc_item_W2, enc_item_b2.reshape(1, D_H),
          enc_item_W3, enc_item_b3.reshape(1, D_H),
          enc_item_W_out, enc_item_b_out.reshape(1, D_H)]

    yu, yi = _tc_dense1(x_user, x_item, scl, pu, pi, gcn1_W_ui, gcn1_W_iu)
    xs1_u = yu.reshape(R, 4, QW).transpose(1, 0, 2).reshape(4 * R, QW)
    xs1_i = yi.reshape(R, 4, QW).transpose(1, 0, 2).reshape(4 * R, QW)

    m = _sc_conv1(xs1_u, xs1_i, edges, zeros16)
    m64 = m.transpose(0, 2, 1, 3).reshape(2, R, D_H)

    w2u = jnp.pad(gcn2_W_ui, ((0, 0), (0, W2 - D_OUT)))
    w2i = jnp.pad(gcn2_W_iu, ((0, 0), (0, W2 - D_OUT)))
    xs2 = _tc_prep2(m64, scl, w2u, w2i)

    o_pre = _sc_conv2(xs2.reshape(2 * R, W2), edges, zeros8)
    o = _tc_final(o_pre, scl)

    return jnp.concatenate([o[1, :N, :D_OUT], o[0, :N, :D_OUT]], axis=0)


# speedup vs baseline: 1.0729x; 1.0729x over previous
import jax
import jax.numpy as jnp
from jax import lax
from jax.experimental import pallas as pl
from jax.experimental.pallas import tpu as pltpu
from jax.experimental.pallas import tpu_sc as plsc

N = 50000
E = 800000
D_IN = 128
D_H = 64
D_OUT = 2
QW = 16
W2 = 8

R = 51200
DUMMY = N
RPT = R // 16
WCH = 400

EP = 802816
EROWS = EP // 128
NCH = 49
CPT = EP // 16 // 128

BLK = 2000
NBLK = N // BLK


def _mesh():
    return plsc.VectorSubcoreMesh(core_axis_name="c", subcore_axis_name="s")


_SC_PARAMS = pltpu.CompilerParams(use_tc_tiling_on_sc=False)


def _edge_sweep(tab, edges, et, poff, spm, rows_v, idxs_v, idxd_v, sem, s):
    def ch(k, carry):
        row0 = s * CPT + k * 8
        pltpu.sync_copy(edges.at[et, 0, pl.ds(row0, 8)], idxs_v)
        pltpu.sync_copy(edges.at[et, 1, pl.ds(row0, 8)], idxd_v)
        offv = jnp.zeros((16,), jnp.int32) + poff

        def ob(r, carry2):
            for l in range(8):
                idxs_v[r, pl.ds(l * 16, 16)] = (
                    idxs_v[r, pl.ds(l * 16, 16)] + offv
                )
            return carry2

        lax.fori_loop(0, 8, ob, 0)
        descs = []
        for j in range(8):
            descs.append(
                pltpu.async_copy(
                    tab.at[idxs_v.at[j]],
                    rows_v.at[pl.ds(j * 128, 128)],
                    sem,
                )
            )
        for j in range(8):
            descs[j].wait()
            pltpu.sync_copy(
                rows_v.at[pl.ds(j * 128, 128)],
                spm.at[idxd_v.at[j]],
                add=True,
            )
        return carry

    lax.fori_loop(0, NCH, ch, 0)


def _zero_spm_rows(zbuf, spm, s):
    for w in range(RPT // WCH):
        pltpu.sync_copy(zbuf, spm.at[pl.ds(s * RPT + w * WCH, WCH)])


def _sc_degrees_body(edges, ones_hbm, zeros_hbm, deg, idx_v, ones_v, zbuf, obuf, spm):
    c = lax.axis_index("c")
    s = lax.axis_index("s")
    pltpu.sync_copy(ones_hbm, ones_v)
    pltpu.sync_copy(zeros_hbm, zbuf)

    for j in range(2):
        _zero_spm_rows(zbuf, spm, s)
        plsc.subcore_barrier()

        def hchunk(k, carry):
            row0 = s * CPT + k * 8
            pltpu.sync_copy(edges.at[c, j, pl.ds(row0, 8)], idx_v)
            for j8 in range(8):
                pltpu.sync_copy(ones_v, spm.at[idx_v.at[j8]], add=True)
            return carry

        lax.fori_loop(0, NCH, hchunk, 0)
        plsc.subcore_barrier()

        for w in range(RPT // WCH):
            r0 = s * RPT + w * WCH
            pltpu.sync_copy(spm.at[pl.ds(r0, WCH)], obuf)
            pltpu.sync_copy(obuf, deg.at[2 * c + j, pl.ds(r0, WCH)])


def _sc_degrees(edges, ones4, zeros4):
    return pl.kernel(
        _sc_degrees_body,
        out_type=jax.ShapeDtypeStruct((4, R, W2), jnp.float32),
        mesh=_mesh(),
        compiler_params=_SC_PARAMS,
        scratch_types=[
            pltpu.VMEM((8, 128), jnp.int32),
            pltpu.VMEM((128, W2), jnp.float32),
            pltpu.VMEM((WCH, W2), jnp.float32),
            pltpu.VMEM((WCH, W2), jnp.float32),
            pltpu.VMEM_SHARED((R, W2), jnp.float32),
        ],
    )(edges, ones4, zeros4)


def _sc_conv1_body(tab_a, tab_b, edges, zeros_hbm, m_out,
                   idxs_v, idxd_v, rows_v, zbuf, spm, sem):
    c = lax.axis_index("c")
    s = lax.axis_index("s")
    pltpu.sync_copy(zeros_hbm, zbuf)

    for et, tab in ((0, tab_a), (1, tab_b)):
        for p in range(2):
            q = 2 * c + p
            _zero_spm_rows(zbuf, spm, s)
            plsc.subcore_barrier()
            _edge_sweep(tab, edges, et, q * R, spm, rows_v, idxs_v, idxd_v, sem, s)
            plsc.subcore_barrier()
            for w in range(RPT // WCH):
                r0 = s * RPT + w * WCH
                pltpu.sync_copy(spm.at[pl.ds(r0, WCH)], rows_v.at[pl.ds(0, WCH)])
                pltpu.sync_copy(
                    rows_v.at[pl.ds(0, WCH)],
                    m_out.at[et, pl.ds(r0, WCH), pl.ds(q * QW, QW)],
                )
            plsc.subcore_barrier()


def _sc_conv1(tab_a, tab_b, edges, zeros16):
    return pl.kernel(
        _sc_conv1_body,
        out_type=jax.ShapeDtypeStruct((2, R, D_H), jnp.float32),
        mesh=_mesh(),
        compiler_params=_SC_PARAMS,
        scratch_types=[
            pltpu.VMEM((8, 128), jnp.int32),
            pltpu.VMEM((8, 128), jnp.int32),
            pltpu.VMEM((1024, QW), jnp.float32),
            pltpu.VMEM((WCH, QW), jnp.float32),
            pltpu.VMEM_SHARED((R, QW), jnp.float32),
            pltpu.SemaphoreType.DMA,
        ],
    )(tab_a, tab_b, edges, zeros16)


def _sc_conv2_body(tab, edges, zeros_hbm, o_out,
                   idxs_v, idxd_v, rows_v, zbuf, spm, sem):
    c = lax.axis_index("c")
    s = lax.axis_index("s")
    pltpu.sync_copy(zeros_hbm, zbuf)
    _zero_spm_rows(zbuf, spm, s)
    plsc.subcore_barrier()
    _edge_sweep(tab, edges, c, c * R, spm, rows_v, idxs_v, idxd_v, sem, s)
    plsc.subcore_barrier()
    for w in range(RPT // WCH):
        r0 = s * RPT + w * WCH
        pltpu.sync_copy(spm.at[pl.ds(r0, WCH)], rows_v.at[pl.ds(0, WCH)])
        pltpu.sync_copy(rows_v.at[pl.ds(0, WCH)], o_out.at[c, pl.ds(r0, WCH)])


def _sc_conv2(tab, edges, zeros4):
    return pl.kernel(
        _sc_conv2_body,
        out_type=jax.ShapeDtypeStruct((2, R, W2), jnp.float32),
        mesh=_mesh(),
        compiler_params=_SC_PARAMS,
        scratch_types=[
            pltpu.VMEM((8, 128), jnp.int32),
            pltpu.VMEM((8, 128), jnp.int32),
            pltpu.VMEM((1024, W2), jnp.float32),
            pltpu.VMEM((WCH, W2), jnp.float32),
            pltpu.VMEM_SHARED((R, W2), jnp.float32),
            pltpu.SemaphoreType.DMA,
        ],
    )(tab, edges, zeros4)


def _scales_body(deg_ref, o_ref):
    o_ref[...] = lax.rsqrt(jnp.maximum(deg_ref[...], 1.0))


def _tc_scales(deg):
    nb = 8
    return pl.pallas_call(
        _scales_body,
        grid=(nb,),
        in_specs=[pl.BlockSpec((4, R // nb, W2), lambda i: (0, i, 0))],
        out_specs=pl.BlockSpec((4, R // nb, W2), lambda i: (0, i, 0)),
        out_shape=jax.ShapeDtypeStruct((4, R, W2), jnp.float32),
    )(deg)


def _encoder(x, p):
    h = jnp.maximum(jnp.dot(x, p[0], preferred_element_type=jnp.float32) + p[1], 0.0)
    for j in range(4):
        h = h + jnp.maximum(
            jnp.dot(h, p[2 + 2 * j], preferred_element_type=jnp.float32)
            + p[3 + 2 * j],
            0.0,
        )
    return jnp.dot(h, p[10], preferred_element_type=jnp.float32) + p[11]


def _dense1_body(*refs):
    xu_ref, xi_ref, scl_ref = refs[0], refs[1], refs[2]
    pu = [r[...] for r in refs[3:15]]
    pi = [r[...] for r in refs[15:27]]
    w1u, w1i = refs[27][...], refs[28][...]
    ou_ref, oi_ref = refs[29], refs[30]

    scl = scl_ref[...]
    hu = _encoder(xu_ref[...], pu)
    hi = _encoder(xi_ref[...], pi)
    yu = jnp.dot(hu, w1u, preferred_element_type=jnp.float32) * scl[:, 0:1]
    yi = jnp.dot(hi, w1i, preferred_element_type=jnp.float32) * scl[:, 2:3]
    for q in range(4):
        ou_ref[q] = yu[:, q * QW:(q + 1) * QW]
        oi_ref[q] = yi[:, q * QW:(q + 1) * QW]


def _full_spec(a):
    nd = a.ndim
    return pl.BlockSpec(a.shape, lambda i, _nd=nd: (0,) * _nd)


def _tc_dense1(x_user, x_item, scl, pu, pi, w1u, w1i):
    in_specs = [
        pl.BlockSpec((BLK, D_IN), lambda i: (i, 0)),
        pl.BlockSpec((BLK, D_IN), lambda i: (i, 0)),
        pl.BlockSpec((BLK, 4), lambda i: (i, 0)),
    ]
    args = [x_user, x_item, scl]
    for a in list(pu) + list(pi) + [w1u, w1i]:
        in_specs.append(_full_spec(a))
        args.append(a)
    out_shape = [jax.ShapeDtypeStruct((4, R, QW), jnp.float32)] * 2
    out_specs = [pl.BlockSpec((4, BLK, QW), lambda i: (0, i, 0))] * 2
    return pl.pallas_call(
        _dense1_body,
        grid=(NBLK,),
        in_specs=in_specs,
        out_specs=out_specs,
        out_shape=out_shape,
    )(*args)


def _prep2_body(m_ref, scl_ref, w2u_ref, w2i_ref, o_ref):
    mm = m_ref[...]
    scl = scl_ref[...]
    h_u1 = jnp.maximum(mm[1] * scl[:, 3:4], 0.0)
    h_i1 = jnp.maximum(mm[0] * scl[:, 1:2], 0.0)
    o_ref[0] = (
        jnp.dot(h_u1, w2u_ref[...], preferred_element_type=jnp.float32)
        * scl[:, 0:1]
    )
    o_ref[1] = (
        jnp.dot(h_i1, w2i_ref[...], preferred_element_type=jnp.float32)
        * scl[:, 2:3]
    )


def _tc_prep2(m64, scl, w2u, w2i):
    return pl.pallas_call(
        _prep2_body,
        grid=(NBLK,),
        in_specs=[
            pl.BlockSpec((2, BLK, D_H), lambda i: (0, i, 0)),
            pl.BlockSpec((BLK, 4), lambda i: (i, 0)),
            _full_spec(w2u),
            _full_spec(w2i),
        ],
        out_specs=pl.BlockSpec((2, BLK, W2), lambda i: (0, i, 0)),
        out_shape=jax.ShapeDtypeStruct((2, R, W2), jnp.float32),
    )(m64, scl, w2u, w2i)


def _final_body(o_ref, scl_ref, out_ref):
    oo = o_ref[...]
    scl = scl_ref[...]
    out_ref[0] = oo[0] * scl[:, 1:2]
    out_ref[1] = oo[1] * scl[:, 3:4]


def _tc_final(o_pre, scl):
    return pl.pallas_call(
        _final_body,
        grid=(NBLK,),
        in_specs=[
            pl.BlockSpec((2, BLK, W2), lambda i: (0, i, 0)),
            pl.BlockSpec((BLK, 4), lambda i: (i, 0)),
        ],
        out_specs=pl.BlockSpec((2, BLK, W2), lambda i: (0, i, 0)),
        out_shape=jax.ShapeDtypeStruct((2, R, W2), jnp.float32),
    )(o_pre, scl)


def _pad_edges(ei):
    pad = jnp.full((EP - E,), DUMMY, jnp.int32)
    return jnp.stack(
        [jnp.concatenate([ei[0], pad]), jnp.concatenate([ei[1], pad])]
    )


def kernel(x_user, x_item, ei_user_item, ei_item_user, enc_user_W_in, enc_user_b_in, enc_user_W0, enc_user_b0, enc_user_W1, enc_user_b1, enc_user_W2, enc_user_b2, enc_user_W3, enc_user_b3, enc_user_W_out, enc_user_b_out, enc_item_W_in, enc_item_b_in, enc_item_W0, enc_item_b0, enc_item_W1, enc_item_b1, enc_item_W2, enc_item_b2, enc_item_W3, enc_item_b3, enc_item_W_out, enc_item_b_out, gcn1_W_ui, gcn1_W_iu, gcn2_W_ui, gcn2_W_iu):
    edges = jnp.stack(
        [_pad_edges(ei_user_item), _pad_edges(ei_item_user)]
    ).reshape(2, 2, EROWS, 128)

    ones4 = jnp.ones((128, W2), jnp.float32)
    zeros4 = jnp.zeros((WCH, W2), jnp.float32)
    zeros16 = jnp.zeros((WCH, QW), jnp.float32)

    deg = _sc_degrees(edges, ones4, zeros4)
    scl = _tc_scales(deg)[:, :, 0].T

    pu = [enc_user_W_in, enc_user_b_in.reshape(1, D_H),
          enc_user_W0, enc_user_b0.reshape(1, D_H),
          enc_user_W1, enc_user_b1.reshape(1, D_H),
          enc_user_W2, enc_user_b2.reshape(1, D_H),
          enc_user_W3, enc_user_b3.reshape(1, D_H),
          enc_user_W_out, enc_user_b_out.reshape(1, D_H)]
    pi = [enc_item_W_in, enc_item_b_in.reshape(1, D_H),
          enc_item_W0, enc_item_b0.reshape(1, D_H),
          enc_item_W1, enc_item_b1.reshape(1, D_H),
          enc_item_W2, enc_item_b2.reshape(1, D_H),
          enc_item_W3, enc_item_b3.reshape(1, D_H),
          enc_item_W_out, enc_item_b_out.reshape(1, D_H)]

    xs1_u, xs1_i = _tc_dense1(x_user, x_item, scl, pu, pi, gcn1_W_ui, gcn1_W_iu)

    m64 = _sc_conv1(
        xs1_u.reshape(4 * R, QW), xs1_i.reshape(4 * R, QW), edges, zeros16
    )

    w2u = jnp.pad(gcn2_W_ui, ((0, 0), (0, W2 - D_OUT)))
    w2i = jnp.pad(gcn2_W_iu, ((0, 0), (0, W2 - D_OUT)))
    xs2 = _tc_prep2(m64, scl, w2u, w2i)

    o_pre = _sc_conv2(xs2.reshape(2 * R, W2), edges, zeros4)
    o = _tc_final(o_pre, scl)

    return jnp.concatenate([o[1, :N, :D_OUT], o[0, :N, :D_OUT]], axis=0)


# speedup vs baseline: 1.2710x; 1.1846x over previous
import jax
import jax.numpy as jnp
from jax import lax
from jax.experimental import pallas as pl
from jax.experimental.pallas import tpu as pltpu
from jax.experimental.pallas import tpu_sc as plsc

N = 50000
E = 800000
D_IN = 128
D_H = 64
D_OUT = 2
QW = 16
W2 = 8

R = 51200
DUMMY = N
RPT = R // 16
WCH = 400

EP = 802816
EROWS = EP // 128
NCH = 49
CPT = EP // 16 // 128

BLK = 2000
NBLK = N // BLK


def _mesh():
    return plsc.VectorSubcoreMesh(core_axis_name="c", subcore_axis_name="s")


_SC_PARAMS = pltpu.CompilerParams(use_tc_tiling_on_sc=False)


def _edge_sweep(tab, edges, et, spm, rows_v, idxs_v, idxd_v, sem, s):
    def ch(k, carry):
        row0 = s * CPT + k * 8
        pltpu.sync_copy(edges.at[et, 0, pl.ds(row0, 8)], idxs_v)
        pltpu.sync_copy(edges.at[et, 1, pl.ds(row0, 8)], idxd_v)
        descs = []
        for j in range(8):
            descs.append(
                pltpu.async_copy(
                    tab.at[idxs_v.at[j]],
                    rows_v.at[pl.ds(j * 128, 128)],
                    sem,
                )
            )
        for j in range(8):
            descs[j].wait()
            pltpu.sync_copy(
                rows_v.at[pl.ds(j * 128, 128)],
                spm.at[idxd_v.at[j]],
                add=True,
            )
        return carry

    lax.fori_loop(0, NCH, ch, 0)


def _zero_spm_rows(zbuf, spm, s):
    for w in range(RPT // WCH):
        pltpu.sync_copy(zbuf, spm.at[pl.ds(s * RPT + w * WCH, WCH)])


def _sc_degrees_body(edges, ones_hbm, zeros_hbm, deg, idx_v, ones_v, zbuf, obuf, spm):
    c = lax.axis_index("c")
    s = lax.axis_index("s")
    pltpu.sync_copy(ones_hbm, ones_v)
    pltpu.sync_copy(zeros_hbm, zbuf)

    for j in range(2):
        _zero_spm_rows(zbuf, spm, s)
        plsc.subcore_barrier()

        def hchunk(k, carry):
            row0 = s * CPT + k * 8
            pltpu.sync_copy(edges.at[c, j, pl.ds(row0, 8)], idx_v)
            for j8 in range(8):
                pltpu.sync_copy(ones_v, spm.at[idx_v.at[j8]], add=True)
            return carry

        lax.fori_loop(0, NCH, hchunk, 0)
        plsc.subcore_barrier()

        for w in range(RPT // WCH):
            r0 = s * RPT + w * WCH
            pltpu.sync_copy(spm.at[pl.ds(r0, WCH)], obuf)
            pltpu.sync_copy(obuf, deg.at[2 * c + j, pl.ds(r0, WCH)])


def _sc_degrees(edges, ones4, zeros4):
    return pl.kernel(
        _sc_degrees_body,
        out_type=jax.ShapeDtypeStruct((4, R, W2), jnp.float32),
        mesh=_mesh(),
        compiler_params=_SC_PARAMS,
        scratch_types=[
            pltpu.VMEM((8, 128), jnp.int32),
            pltpu.VMEM((128, W2), jnp.float32),
            pltpu.VMEM((WCH, W2), jnp.float32),
            pltpu.VMEM((WCH, W2), jnp.float32),
            pltpu.VMEM_SHARED((R, W2), jnp.float32),
        ],
    )(edges, ones4, zeros4)


def _sc_conv1_body(tab_a, tab_b, edges, zeros_hbm, m_out,
                   idxs_v, idxd_v, rows_v, zbuf, spm, sem):
    c = lax.axis_index("c")
    s = lax.axis_index("s")
    pltpu.sync_copy(zeros_hbm, zbuf)

    for et, tab in ((0, tab_a), (1, tab_b)):
        for p in range(2):
            q = 2 * c + p
            _zero_spm_rows(zbuf, spm, s)
            plsc.subcore_barrier()
            _edge_sweep(tab.at[q], edges, et, spm, rows_v, idxs_v, idxd_v, sem, s)
            plsc.subcore_barrier()
            for w in range(RPT // WCH):
                r0 = s * RPT + w * WCH
                pltpu.sync_copy(spm.at[pl.ds(r0, WCH)], rows_v.at[pl.ds(0, WCH)])
                pltpu.sync_copy(
                    rows_v.at[pl.ds(0, WCH)],
                    m_out.at[et, pl.ds(r0, WCH), pl.ds(q * QW, QW)],
                )
            plsc.subcore_barrier()


def _sc_conv1(tab_a, tab_b, edges, zeros16):
    return pl.kernel(
        _sc_conv1_body,
        out_type=jax.ShapeDtypeStruct((2, R, D_H), jnp.float32),
        mesh=_mesh(),
        compiler_params=_SC_PARAMS,
        scratch_types=[
            pltpu.VMEM((8, 128), jnp.int32),
            pltpu.VMEM((8, 128), jnp.int32),
            pltpu.VMEM((1024, QW), jnp.float32),
            pltpu.VMEM((WCH, QW), jnp.float32),
            pltpu.VMEM_SHARED((R, QW), jnp.float32),
            pltpu.SemaphoreType.DMA,
        ],
    )(tab_a, tab_b, edges, zeros16)


def _sc_conv2_body(tab, edges, zeros_hbm, o_out,
                   idxs_v, idxd_v, rows_v, zbuf, spm, sem):
    c = lax.axis_index("c")
    s = lax.axis_index("s")
    pltpu.sync_copy(zeros_hbm, zbuf)
    _zero_spm_rows(zbuf, spm, s)
    plsc.subcore_barrier()
    _edge_sweep(tab.at[c], edges, c, spm, rows_v, idxs_v, idxd_v, sem, s)
    plsc.subcore_barrier()
    for w in range(RPT // WCH):
        r0 = s * RPT + w * WCH
        pltpu.sync_copy(spm.at[pl.ds(r0, WCH)], rows_v.at[pl.ds(0, WCH)])
        pltpu.sync_copy(rows_v.at[pl.ds(0, WCH)], o_out.at[c, pl.ds(r0, WCH)])


def _sc_conv2(tab, edges, zeros4):
    return pl.kernel(
        _sc_conv2_body,
        out_type=jax.ShapeDtypeStruct((2, R, W2), jnp.float32),
        mesh=_mesh(),
        compiler_params=_SC_PARAMS,
        scratch_types=[
            pltpu.VMEM((8, 128), jnp.int32),
            pltpu.VMEM((8, 128), jnp.int32),
            pltpu.VMEM((1024, W2), jnp.float32),
            pltpu.VMEM((WCH, W2), jnp.float32),
            pltpu.VMEM_SHARED((R, W2), jnp.float32),
            pltpu.SemaphoreType.DMA,
        ],
    )(tab, edges, zeros4)


def _encoder(x, p):
    h = jnp.maximum(jnp.dot(x, p[0], preferred_element_type=jnp.float32) + p[1], 0.0)
    for j in range(4):
        h = h + jnp.maximum(
            jnp.dot(h, p[2 + 2 * j], preferred_element_type=jnp.float32)
            + p[3 + 2 * j],
            0.0,
        )
    return jnp.dot(h, p[10], preferred_element_type=jnp.float32) + p[11]


def _dense1_body(*refs):
    xu_ref, xi_ref, deg_ref = refs[0], refs[1], refs[2]
    pu = [r[...] for r in refs[3:15]]
    pi = [r[...] for r in refs[15:27]]
    w1u, w1i = refs[27][...], refs[28][...]
    ou_ref, oi_ref = refs[29], refs[30]

    dd = deg_ref[...]
    hu = _encoder(xu_ref[...], pu)
    hi = _encoder(xi_ref[...], pi)
    su = lax.rsqrt(jnp.maximum(dd[0, :, 0:1], 1.0))
    si = lax.rsqrt(jnp.maximum(dd[2, :, 0:1], 1.0))
    yu = jnp.dot(hu, w1u, preferred_element_type=jnp.float32) * su
    yi = jnp.dot(hi, w1i, preferred_element_type=jnp.float32) * si
    for q in range(4):
        ou_ref[q] = yu[:, q * QW:(q + 1) * QW]
        oi_ref[q] = yi[:, q * QW:(q + 1) * QW]


def _full_spec(a):
    nd = a.ndim
    return pl.BlockSpec(a.shape, lambda i, _nd=nd: (0,) * _nd)


def _tc_dense1(x_user, x_item, deg, pu, pi, w1u, w1i):
    in_specs = [
        pl.BlockSpec((BLK, D_IN), lambda i: (i, 0)),
        pl.BlockSpec((BLK, D_IN), lambda i: (i, 0)),
        pl.BlockSpec((4, BLK, W2), lambda i: (0, i, 0)),
    ]
    args = [x_user, x_item, deg]
    for a in list(pu) + list(pi) + [w1u, w1i]:
        in_specs.append(_full_spec(a))
        args.append(a)
    out_shape = [jax.ShapeDtypeStruct((4, R, QW), jnp.float32)] * 2
    out_specs = [pl.BlockSpec((4, BLK, QW), lambda i: (0, i, 0))] * 2
    return pl.pallas_call(
        _dense1_body,
        grid=(NBLK,),
        in_specs=in_specs,
        out_specs=out_specs,
        out_shape=out_shape,
    )(*args)


def _prep2_body(m_ref, deg_ref, w2u_ref, w2i_ref, o_ref):
    mm = m_ref[...]
    dd = deg_ref[...]
    h_u1 = jnp.maximum(mm[1] * lax.rsqrt(jnp.maximum(dd[3, :, 0:1], 1.0)), 0.0)
    h_i1 = jnp.maximum(mm[0] * lax.rsqrt(jnp.maximum(dd[1, :, 0:1], 1.0)), 0.0)
    o_ref[0] = (
        jnp.dot(h_u1, w2u_ref[...], preferred_element_type=jnp.float32)
        * lax.rsqrt(jnp.maximum(dd[0, :, 0:1], 1.0))
    )
    o_ref[1] = (
        jnp.dot(h_i1, w2i_ref[...], preferred_element_type=jnp.float32)
        * lax.rsqrt(jnp.maximum(dd[2, :, 0:1], 1.0))
    )


def _tc_prep2(m64, deg, w2u, w2i):
    return pl.pallas_call(
        _prep2_body,
        grid=(NBLK,),
        in_specs=[
            pl.BlockSpec((2, BLK, D_H), lambda i: (0, i, 0)),
            pl.BlockSpec((4, BLK, W2), lambda i: (0, i, 0)),
            _full_spec(w2u),
            _full_spec(w2i),
        ],
        out_specs=pl.BlockSpec((2, BLK, W2), lambda i: (0, i, 0)),
        out_shape=jax.ShapeDtypeStruct((2, R, W2), jnp.float32),
    )(m64, deg, w2u, w2i)


def _final_body(o_ref, deg_ref, out_ref):
    oo = o_ref[...]
    dd = deg_ref[...]
    out_ref[0] = oo[0] * lax.rsqrt(jnp.maximum(dd[1, :, 0:1], 1.0))
    out_ref[1] = oo[1] * lax.rsqrt(jnp.maximum(dd[3, :, 0:1], 1.0))


def _tc_final(o_pre, deg):
    return pl.pallas_call(
        _final_body,
        grid=(NBLK,),
        in_specs=[
            pl.BlockSpec((2, BLK, W2), lambda i: (0, i, 0)),
            pl.BlockSpec((4, BLK, W2), lambda i: (0, i, 0)),
        ],
        out_specs=pl.BlockSpec((2, BLK, W2), lambda i: (0, i, 0)),
        out_shape=jax.ShapeDtypeStruct((2, R, W2), jnp.float32),
    )(o_pre, deg)


def _pad_edges(ei):
    pad = jnp.full((EP - E,), DUMMY, jnp.int32)
    return jnp.stack(
        [jnp.concatenate([ei[0], pad]), jnp.concatenate([ei[1], pad])]
    )


def kernel(x_user, x_item, ei_user_item, ei_item_user, enc_user_W_in, enc_user_b_in, enc_user_W0, enc_user_b0, enc_user_W1, enc_user_b1, enc_user_W2, enc_user_b2, enc_user_W3, enc_user_b3, enc_user_W_out, enc_user_b_out, enc_item_W_in, enc_item_b_in, enc_item_W0, enc_item_b0, enc_item_W1, enc_item_b1, enc_item_W2, enc_item_b2, enc_item_W3, enc_item_b3, enc_item_W_out, enc_item_b_out, gcn1_W_ui, gcn1_W_iu, gcn2_W_ui, gcn2_W_iu):
    edges = jnp.stack(
        [_pad_edges(ei_user_item), _pad_edges(ei_item_user)]
    ).reshape(2, 2, EROWS, 128)

    ones4 = jnp.ones((128, W2), jnp.float32)
    zeros4 = jnp.zeros((WCH, W2), jnp.float32)
    zeros16 = jnp.zeros((WCH, QW), jnp.float32)

    deg = _sc_degrees(edges, ones4, zeros4)

    pu = [enc_user_W_in, enc_user_b_in.reshape(1, D_H),
          enc_user_W0, enc_user_b0.reshape(1, D_H),
          enc_user_W1, enc_user_b1.reshape(1, D_H),
          enc_user_W2, enc_user_b2.reshape(1, D_H),
          enc_user_W3, enc_user_b3.reshape(1, D_H),
          enc_user_W_out, enc_user_b_out.reshape(1, D_H)]
    pi = [enc_item_W_in, enc_item_b_in.reshape(1, D_H),
          enc_item_W0, enc_item_b0.reshape(1, D_H),
          enc_item_W1, enc_item_b1.reshape(1, D_H),
          enc_item_W2, enc_item_b2.reshape(1, D_H),
          enc_item_W3, enc_item_b3.reshape(1, D_H),
          enc_item_W_out, enc_item_b_out.reshape(1, D_H)]

    xs1_u, xs1_i = _tc_dense1(x_user, x_item, deg, pu, pi, gcn1_W_ui, gcn1_W_iu)

    m64 = _sc_conv1(xs1_u, xs1_i, edges, zeros16)

    w2u = jnp.pad(gcn2_W_ui, ((0, 0), (0, W2 - D_OUT)))
    w2i = jnp.pad(gcn2_W_iu, ((0, 0), (0, W2 - D_OUT)))
    xs2 = _tc_prep2(m64, deg, w2u, w2i)

    o_pre = _sc_conv2(xs2, edges, zeros4)
    o = _tc_final(o_pre, deg)

    return jnp.concatenate([o[1, :N, :D_OUT], o[0, :N, :D_OUT]], axis=0)


# speedup vs baseline: 1.5232x; 1.1984x over previous
import jax
import jax.numpy as jnp
from jax import lax
from jax.experimental import pallas as pl
from jax.experimental.pallas import tpu as pltpu
from jax.experimental.pallas import tpu_sc as plsc

N = 50000
E = 800000
D_IN = 128
D_H = 64
D_OUT = 2
QW = 16
W2 = 8

R = 51200
DUMMY = N
RPT = R // 16
WCH = 400

EP = 802816
EROWS = EP // 128
NCH = 49
CPT = EP // 16 // 128

BLK = 2000
NBLK = N // BLK


def _mesh():
    return plsc.VectorSubcoreMesh(core_axis_name="c", subcore_axis_name="s")


_SC_PARAMS = pltpu.CompilerParams(use_tc_tiling_on_sc=False)


def _edge_sweep(tab, edges, et, spm, rows_v, idxs_v, idxd_v, sem, isem, s):
    base = s * CPT
    pltpu.sync_copy(edges.at[et, 0, pl.ds(base, 8)], idxs_v.at[0])
    pltpu.sync_copy(edges.at[et, 1, pl.ds(base, 8)], idxd_v.at[0])

    def ch(k, carry):
        cur = lax.rem(k, 2)
        nxt = lax.rem(k + 1, 2)

        @pl.when(k > 0)
        def _drain():
            row0 = base + k * 8
            pltpu.make_async_copy(
                edges.at[et, 0, pl.ds(row0, 8)], idxs_v.at[cur], isem
            ).wait()
            pltpu.make_async_copy(
                edges.at[et, 1, pl.ds(row0, 8)], idxd_v.at[cur], isem
            ).wait()

        @pl.when(k + 1 < NCH)
        def _prefetch():
            row1 = base + (k + 1) * 8
            pltpu.async_copy(edges.at[et, 0, pl.ds(row1, 8)], idxs_v.at[nxt], isem)
            pltpu.async_copy(edges.at[et, 1, pl.ds(row1, 8)], idxd_v.at[nxt], isem)

        descs = []
        for j in range(8):
            descs.append(
                pltpu.async_copy(
                    tab.at[idxs_v.at[cur, j]],
                    rows_v.at[pl.ds(j * 128, 128)],
                    sem,
                )
            )
        for j in range(8):
            descs[j].wait()
            pltpu.sync_copy(
                rows_v.at[pl.ds(j * 128, 128)],
                spm.at[idxd_v.at[cur, j]],
                add=True,
            )
        return carry

    lax.fori_loop(0, NCH, ch, 0)


def _zero_spm_rows(zbuf, spm, s):
    for w in range(RPT // WCH):
        pltpu.sync_copy(zbuf, spm.at[pl.ds(s * RPT + w * WCH, WCH)])


def _sc_degrees_body(edges, ones_hbm, zeros_hbm, deg, idx_v, ones_v, zbuf, obuf, spm, isem):
    c = lax.axis_index("c")
    s = lax.axis_index("s")
    base = s * CPT
    pltpu.sync_copy(ones_hbm, ones_v)
    pltpu.sync_copy(zeros_hbm, zbuf)

    for j in range(2):
        _zero_spm_rows(zbuf, spm, s)
        plsc.subcore_barrier()
        pltpu.sync_copy(edges.at[c, j, pl.ds(base, 8)], idx_v.at[0])

        def hchunk(k, carry):
            cur = lax.rem(k, 2)
            nxt = lax.rem(k + 1, 2)

            @pl.when(k > 0)
            def _drain():
                row0 = base + k * 8
                pltpu.make_async_copy(
                    edges.at[c, j, pl.ds(row0, 8)], idx_v.at[cur], isem
                ).wait()

            @pl.when(k + 1 < NCH)
            def _prefetch():
                row1 = base + (k + 1) * 8
                pltpu.async_copy(edges.at[c, j, pl.ds(row1, 8)], idx_v.at[nxt], isem)

            for j8 in range(8):
                pltpu.sync_copy(ones_v, spm.at[idx_v.at[cur, j8]], add=True)
            return carry

        lax.fori_loop(0, NCH, hchunk, 0)
        plsc.subcore_barrier()

        for w in range(RPT // WCH):
            r0 = s * RPT + w * WCH
            pltpu.sync_copy(spm.at[pl.ds(r0, WCH)], obuf)
            pltpu.sync_copy(obuf, deg.at[2 * c + j, pl.ds(r0, WCH)])


def _sc_degrees(edges, ones4, zeros4):
    return pl.kernel(
        _sc_degrees_body,
        out_type=jax.ShapeDtypeStruct((4, R, W2), jnp.float32),
        mesh=_mesh(),
        compiler_params=_SC_PARAMS,
        scratch_types=[
            pltpu.VMEM((2, 8, 128), jnp.int32),
            pltpu.VMEM((128, W2), jnp.float32),
            pltpu.VMEM((WCH, W2), jnp.float32),
            pltpu.VMEM((WCH, W2), jnp.float32),
            pltpu.VMEM_SHARED((R, W2), jnp.float32),
            pltpu.SemaphoreType.DMA,
        ],
    )(edges, ones4, zeros4)


def _sc_conv1_body(tab_a, tab_b, edges, zeros_hbm, m_out,
                   idxs_v, idxd_v, rows_v, zbuf, spm, sem, isem):
    c = lax.axis_index("c")
    s = lax.axis_index("s")
    pltpu.sync_copy(zeros_hbm, zbuf)

    for et, tab in ((0, tab_a), (1, tab_b)):
        for p in range(2):
            q = 2 * c + p
            _zero_spm_rows(zbuf, spm, s)
            plsc.subcore_barrier()
            _edge_sweep(tab.at[q], edges, et, spm, rows_v, idxs_v, idxd_v, sem, isem, s)
            plsc.subcore_barrier()
            for w in range(RPT // WCH):
                r0 = s * RPT + w * WCH
                pltpu.sync_copy(spm.at[pl.ds(r0, WCH)], rows_v.at[pl.ds(0, WCH)])
                pltpu.sync_copy(
                    rows_v.at[pl.ds(0, WCH)],
                    m_out.at[et, pl.ds(r0, WCH), pl.ds(q * QW, QW)],
                )
            plsc.subcore_barrier()


def _sc_conv1(tab_a, tab_b, edges, zeros16):
    return pl.kernel(
        _sc_conv1_body,
        out_type=jax.ShapeDtypeStruct((2, R, D_H), jnp.float32),
        mesh=_mesh(),
        compiler_params=_SC_PARAMS,
        scratch_types=[
            pltpu.VMEM((2, 8, 128), jnp.int32),
            pltpu.VMEM((2, 8, 128), jnp.int32),
            pltpu.VMEM((1024, QW), jnp.float32),
            pltpu.VMEM((WCH, QW), jnp.float32),
            pltpu.VMEM_SHARED((R, QW), jnp.float32),
            pltpu.SemaphoreType.DMA,
            pltpu.SemaphoreType.DMA,
        ],
    )(tab_a, tab_b, edges, zeros16)


def _sc_conv2_body(tab, edges, zeros_hbm, o_out,
                   idxs_v, idxd_v, rows_v, zbuf, spm, sem, isem):
    c = lax.axis_index("c")
    s = lax.axis_index("s")
    pltpu.sync_copy(zeros_hbm, zbuf)
    _zero_spm_rows(zbuf, spm, s)
    plsc.subcore_barrier()
    _edge_sweep(tab.at[c], edges, c, spm, rows_v, idxs_v, idxd_v, sem, isem, s)
    plsc.subcore_barrier()
    for w in range(RPT // WCH):
        r0 = s * RPT + w * WCH
        pltpu.sync_copy(spm.at[pl.ds(r0, WCH)], rows_v.at[pl.ds(0, WCH)])
        pltpu.sync_copy(rows_v.at[pl.ds(0, WCH)], o_out.at[c, pl.ds(r0, WCH)])


def _sc_conv2(tab, edges, zeros4):
    return pl.kernel(
        _sc_conv2_body,
        out_type=jax.ShapeDtypeStruct((2, R, W2), jnp.float32),
        mesh=_mesh(),
        compiler_params=_SC_PARAMS,
        scratch_types=[
            pltpu.VMEM((2, 8, 128), jnp.int32),
            pltpu.VMEM((2, 8, 128), jnp.int32),
            pltpu.VMEM((1024, W2), jnp.float32),
            pltpu.VMEM((WCH, W2), jnp.float32),
            pltpu.VMEM_SHARED((R, W2), jnp.float32),
            pltpu.SemaphoreType.DMA,
            pltpu.SemaphoreType.DMA,
        ],
    )(tab, edges, zeros4)


def _encoder(x, p):
    h = jnp.maximum(jnp.dot(x, p[0], preferred_element_type=jnp.float32) + p[1], 0.0)
    for j in range(4):
        h = h + jnp.maximum(
            jnp.dot(h, p[2 + 2 * j], preferred_element_type=jnp.float32)
            + p[3 + 2 * j],
            0.0,
        )
    return jnp.dot(h, p[10], preferred_element_type=jnp.float32) + p[11]


def _dense1_body(*refs):
    xu_ref, xi_ref, deg_ref = refs[0], refs[1], refs[2]
    pu = [r[...] for r in refs[3:15]]
    pi = [r[...] for r in refs[15:27]]
    w1u, w1i = refs[27][...], refs[28][...]
    ou_ref, oi_ref = refs[29], refs[30]

    dd = deg_ref[...]
    hu = _encoder(xu_ref[...], pu)
    hi = _encoder(xi_ref[...], pi)
    su = lax.rsqrt(jnp.maximum(dd[0, :, 0:1], 1.0))
    si = lax.rsqrt(jnp.maximum(dd[2, :, 0:1], 1.0))
    yu = jnp.dot(hu, w1u, preferred_element_type=jnp.float32) * su
    yi = jnp.dot(hi, w1i, preferred_element_type=jnp.float32) * si
    for q in range(4):
        ou_ref[q] = yu[:, q * QW:(q + 1) * QW]
        oi_ref[q] = yi[:, q * QW:(q + 1) * QW]


def _full_spec(a):
    nd = a.ndim
    return pl.BlockSpec(a.shape, lambda i, _nd=nd: (0,) * _nd)


def _tc_dense1(x_user, x_item, deg, pu, pi, w1u, w1i):
    in_specs = [
        pl.BlockSpec((BLK, D_IN), lambda i: (i, 0)),
        pl.BlockSpec((BLK, D_IN), lambda i: (i, 0)),
        pl.BlockSpec((4, BLK, W2), lambda i: (0, i, 0)),
    ]
    args = [x_user, x_item, deg]
    for a in list(pu) + list(pi) + [w1u, w1i]:
        in_specs.append(_full_spec(a))
        args.append(a)
    out_shape = [jax.ShapeDtypeStruct((4, R, QW), jnp.float32)] * 2
    out_specs = [pl.BlockSpec((4, BLK, QW), lambda i: (0, i, 0))] * 2
    return pl.pallas_call(
        _dense1_body,
        grid=(NBLK,),
        in_specs=in_specs,
        out_specs=out_specs,
        out_shape=out_shape,
    )(*args)


def _prep2_body(m_ref, deg_ref, w2u_ref, w2i_ref, o_ref):
    mm = m_ref[...]
    dd = deg_ref[...]
    h_u1 = jnp.maximum(mm[1] * lax.rsqrt(jnp.maximum(dd[3, :, 0:1], 1.0)), 0.0)
    h_i1 = jnp.maximum(mm[0] * lax.rsqrt(jnp.maximum(dd[1, :, 0:1], 1.0)), 0.0)
    o_ref[0] = (
        jnp.dot(h_u1, w2u_ref[...], preferred_element_type=jnp.float32)
        * lax.rsqrt(jnp.maximum(dd[0, :, 0:1], 1.0))
    )
    o_ref[1] = (
        jnp.dot(h_i1, w2i_ref[...], preferred_element_type=jnp.float32)
        * lax.rsqrt(jnp.maximum(dd[2, :, 0:1], 1.0))
    )


def _tc_prep2(m64, deg, w2u, w2i):
    return pl.pallas_call(
        _prep2_body,
        grid=(NBLK,),
        in_specs=[
            pl.BlockSpec((2, BLK, D_H), lambda i: (0, i, 0)),
            pl.BlockSpec((4, BLK, W2), lambda i: (0, i, 0)),
            _full_spec(w2u),
            _full_spec(w2i),
        ],
        out_specs=pl.BlockSpec((2, BLK, W2), lambda i: (0, i, 0)),
        out_shape=jax.ShapeDtypeStruct((2, R, W2), jnp.float32),
    )(m64, deg, w2u, w2i)


def _final_body(o_ref, deg_ref, out_ref):
    oo = o_ref[...]
    dd = deg_ref[...]
    out_ref[0] = oo[0] * lax.rsqrt(jnp.maximum(dd[1, :, 0:1], 1.0))
    out_ref[1] = oo[1] * lax.rsqrt(jnp.maximum(dd[3, :, 0:1], 1.0))


def _tc_final(o_pre, deg):
    return pl.pallas_call(
        _final_body,
        grid=(NBLK,),
        in_specs=[
            pl.BlockSpec((2, BLK, W2), lambda i: (0, i, 0)),
            pl.BlockSpec((4, BLK, W2), lambda i: (0, i, 0)),
        ],
        out_specs=pl.BlockSpec((2, BLK, W2), lambda i: (0, i, 0)),
        out_shape=jax.ShapeDtypeStruct((2, R, W2), jnp.float32),
    )(o_pre, deg)


def _pad_edges(ei):
    pad = jnp.full((EP - E,), DUMMY, jnp.int32)
    return jnp.stack(
        [jnp.concatenate([ei[0], pad]), jnp.concatenate([ei[1], pad])]
    )


def kernel(x_user, x_item, ei_user_item, ei_item_user, enc_user_W_in, enc_user_b_in, enc_user_W0, enc_user_b0, enc_user_W1, enc_user_b1, enc_user_W2, enc_user_b2, enc_user_W3, enc_user_b3, enc_user_W_out, enc_user_b_out, enc_item_W_in, enc_item_b_in, enc_item_W0, enc_item_b0, enc_item_W1, enc_item_b1, enc_item_W2, enc_item_b2, enc_item_W3, enc_item_b3, enc_item_W_out, enc_item_b_out, gcn1_W_ui, gcn1_W_iu, gcn2_W_ui, gcn2_W_iu):
    edges = jnp.stack(
        [_pad_edges(ei_user_item), _pad_edges(ei_item_user)]
    ).reshape(2, 2, EROWS, 128)

    ones4 = jnp.ones((128, W2), jnp.float32)
    zeros4 = jnp.zeros((WCH, W2), jnp.float32)
    zeros16 = jnp.zeros((WCH, QW), jnp.float32)

    deg = _sc_degrees(edges, ones4, zeros4)

    pu = [enc_user_W_in, enc_user_b_in.reshape(1, D_H),
          enc_user_W0, enc_user_b0.reshape(1, D_H),
          enc_user_W1, enc_user_b1.reshape(1, D_H),
          enc_user_W2, enc_user_b2.reshape(1, D_H),
          enc_user_W3, enc_user_b3.reshape(1, D_H),
          enc_user_W_out, enc_user_b_out.reshape(1, D_H)]
    pi = [enc_item_W_in, enc_item_b_in.reshape(1, D_H),
          enc_item_W0, enc_item_b0.reshape(1, D_H),
          enc_item_W1, enc_item_b1.reshape(1, D_H),
          enc_item_W2, enc_item_b2.reshape(1, D_H),
          enc_item_W3, enc_item_b3.reshape(1, D_H),
          enc_item_W_out, enc_item_b_out.reshape(1, D_H)]

    xs1_u, xs1_i = _tc_dense1(x_user, x_item, deg, pu, pi, gcn1_W_ui, gcn1_W_iu)

    m64 = _sc_conv1(xs1_u, xs1_i, edges, zeros16)

    w2u = jnp.pad(gcn2_W_ui, ((0, 0), (0, W2 - D_OUT)))
    w2i = jnp.pad(gcn2_W_iu, ((0, 0), (0, W2 - D_OUT)))
    xs2 = _tc_prep2(m64, deg, w2u, w2i)

    o_pre = _sc_conv2(xs2, edges, zeros4)
    o = _tc_final(o_pre, deg)

    return jnp.concatenate([o[1, :N, :D_OUT], o[0, :N, :D_OUT]], axis=0)


# speedup vs baseline: 1.8476x; 1.2130x over previous
import jax
import jax.numpy as jnp
from jax import lax
from jax.experimental import pallas as pl
from jax.experimental.pallas import tpu as pltpu
from jax.experimental.pallas import tpu_sc as plsc

N = 50000
E = 800000
D_IN = 128
D_H = 64
D_OUT = 2
QW = 16
W2 = 8

R = 51200
DUMMY = N
RPT = R // 16
WCH = 400

EP = 802816
EROWS = EP // 128
NCH = 49
CPT = EP // 16 // 128

BLK = 2000
NBLK = N // BLK


def _mesh():
    return plsc.VectorSubcoreMesh(core_axis_name="c", subcore_axis_name="s")


_SC_PARAMS = pltpu.CompilerParams(use_tc_tiling_on_sc=False)


def _edge_sweep(tab, edges, et, spm, rows_v, idxs_v, idxd_v, gsem0, gsem1, isem, s):
    base = s * CPT
    gsems = (gsem0, gsem1)

    def fire_gathers(ib, gsem):
        for j in range(8):
            pltpu.async_copy(
                tab.at[idxs_v.at[ib, j]],
                rows_v.at[ib % 2, pl.ds(j * 128, 128)],
                gsem,
            )

    def prefetch_idx(k, ib):
        row0 = base + k * 8
        pltpu.async_copy(edges.at[et, 0, pl.ds(row0, 8)], idxs_v.at[ib], isem)
        pltpu.async_copy(edges.at[et, 1, pl.ds(row0, 8)], idxd_v.at[ib], isem)

    def drain_idx(k, ib):
        row0 = base + k * 8
        pltpu.make_async_copy(
            edges.at[et, 0, pl.ds(row0, 8)], idxs_v.at[ib], isem
        ).wait()
        pltpu.make_async_copy(
            edges.at[et, 1, pl.ds(row0, 8)], idxd_v.at[ib], isem
        ).wait()

    pltpu.sync_copy(edges.at[et, 0, pl.ds(base, 8)], idxs_v.at[0])
    pltpu.sync_copy(edges.at[et, 1, pl.ds(base, 8)], idxd_v.at[0])
    fire_gathers(0, gsem0)
    prefetch_idx(1, 1)
    prefetch_idx(2, 2)

    def _maybe(pred, fn):
        if isinstance(pred, bool):
            if pred:
                fn()
        else:
            pl.when(pred)(fn)

    def process(k, i4):
        def _next():
            drain_idx(k + 1, (i4 + 1) % 4)
            fire_gathers((i4 + 1) % 4, gsems[(i4 + 1) % 2])

        _maybe(k + 1 < NCH, _next)
        _maybe(k + 3 < NCH, lambda: prefetch_idx(k + 3, (i4 + 3) % 4))

        cur = i4 % 2
        for j in range(8):
            pltpu.make_async_copy(
                tab.at[idxs_v.at[i4, j]],
                rows_v.at[cur, pl.ds(j * 128, 128)],
                gsems[cur],
            ).wait()
            pltpu.sync_copy(
                rows_v.at[cur, pl.ds(j * 128, 128)],
                spm.at[idxd_v.at[i4, j]],
                add=True,
            )

    def quad(t, carry):
        for u in range(4):
            process(4 * t + u, u)
        return carry

    lax.fori_loop(0, NCH // 4, quad, 0)
    process(NCH - 1, (NCH - 1) % 4)


def _zero_spm_rows(zbuf, spm, s):
    for w in range(RPT // WCH):
        pltpu.sync_copy(zbuf, spm.at[pl.ds(s * RPT + w * WCH, WCH)])


def _sc_degrees_body(edges, ones_hbm, zeros_hbm, deg, idx_v, ones_v, zbuf, obuf, spm, isem):
    c = lax.axis_index("c")
    s = lax.axis_index("s")
    base = s * CPT
    pltpu.sync_copy(ones_hbm, ones_v)
    pltpu.sync_copy(zeros_hbm, zbuf)

    for j in range(2):
        _zero_spm_rows(zbuf, spm, s)
        plsc.subcore_barrier()
        pltpu.sync_copy(edges.at[c, j, pl.ds(base, 8)], idx_v.at[0])

        def hchunk(k, carry):
            cur = lax.rem(k, 2)
            nxt = lax.rem(k + 1, 2)

            @pl.when(k > 0)
            def _drain():
                row0 = base + k * 8
                pltpu.make_async_copy(
                    edges.at[c, j, pl.ds(row0, 8)], idx_v.at[cur], isem
                ).wait()

            @pl.when(k + 1 < NCH)
            def _prefetch():
                row1 = base + (k + 1) * 8
                pltpu.async_copy(edges.at[c, j, pl.ds(row1, 8)], idx_v.at[nxt], isem)

            for j8 in range(8):
                pltpu.sync_copy(ones_v, spm.at[idx_v.at[cur, j8]], add=True)
            return carry

        lax.fori_loop(0, NCH, hchunk, 0)
        plsc.subcore_barrier()

        for w in range(RPT // WCH):
            r0 = s * RPT + w * WCH
            pltpu.sync_copy(spm.at[pl.ds(r0, WCH)], obuf)
            pltpu.sync_copy(obuf, deg.at[2 * c + j, pl.ds(r0, WCH)])


def _sc_degrees(edges, ones4, zeros4):
    return pl.kernel(
        _sc_degrees_body,
        out_type=jax.ShapeDtypeStruct((4, R, W2), jnp.float32),
        mesh=_mesh(),
        compiler_params=_SC_PARAMS,
        scratch_types=[
            pltpu.VMEM((2, 8, 128), jnp.int32),
            pltpu.VMEM((128, W2), jnp.float32),
            pltpu.VMEM((WCH, W2), jnp.float32),
            pltpu.VMEM((WCH, W2), jnp.float32),
            pltpu.VMEM_SHARED((R, W2), jnp.float32),
            pltpu.SemaphoreType.DMA,
        ],
    )(edges, ones4, zeros4)


def _sc_conv1_body(tab_a, tab_b, edges, zeros_hbm, m_out,
                   idxs_v, idxd_v, rows_v, zbuf, spm, gsem0, gsem1, isem):
    c = lax.axis_index("c")
    s = lax.axis_index("s")
    pltpu.sync_copy(zeros_hbm, zbuf)

    for et, tab in ((0, tab_a), (1, tab_b)):
        for p in range(2):
            q = 2 * c + p
            _zero_spm_rows(zbuf, spm, s)
            plsc.subcore_barrier()
            _edge_sweep(tab.at[q], edges, et, spm, rows_v, idxs_v, idxd_v,
                        gsem0, gsem1, isem, s)
            plsc.subcore_barrier()
            for w in range(RPT // WCH):
                r0 = s * RPT + w * WCH
                pltpu.sync_copy(spm.at[pl.ds(r0, WCH)], rows_v.at[0, pl.ds(0, WCH)])
                pltpu.sync_copy(
                    rows_v.at[0, pl.ds(0, WCH)],
                    m_out.at[et, pl.ds(r0, WCH), pl.ds(q * QW, QW)],
                )
            plsc.subcore_barrier()


def _sc_conv1(tab_a, tab_b, edges, zeros16):
    return pl.kernel(
        _sc_conv1_body,
        out_type=jax.ShapeDtypeStruct((2, R, D_H), jnp.float32),
        mesh=_mesh(),
        compiler_params=_SC_PARAMS,
        scratch_types=[
            pltpu.VMEM((4, 8, 128), jnp.int32),
            pltpu.VMEM((4, 8, 128), jnp.int32),
            pltpu.VMEM((2, 1024, QW), jnp.float32),
            pltpu.VMEM((WCH, QW), jnp.float32),
            pltpu.VMEM_SHARED((R, QW), jnp.float32),
            pltpu.SemaphoreType.DMA,
            pltpu.SemaphoreType.DMA,
            pltpu.SemaphoreType.DMA,
        ],
    )(tab_a, tab_b, edges, zeros16)


def _sc_conv2_body(tab, edges, zeros_hbm, o_out,
                   idxs_v, idxd_v, rows_v, zbuf, spm, gsem0, gsem1, isem):
    c = lax.axis_index("c")
    s = lax.axis_index("s")
    pltpu.sync_copy(zeros_hbm, zbuf)
    _zero_spm_rows(zbuf, spm, s)
    plsc.subcore_barrier()
    _edge_sweep(tab.at[c], edges, c, spm, rows_v, idxs_v, idxd_v,
                gsem0, gsem1, isem, s)
    plsc.subcore_barrier()
    for w in range(RPT // WCH):
        r0 = s * RPT + w * WCH
        pltpu.sync_copy(spm.at[pl.ds(r0, WCH)], rows_v.at[0, pl.ds(0, WCH)])
        pltpu.sync_copy(rows_v.at[0, pl.ds(0, WCH)], o_out.at[c, pl.ds(r0, WCH)])


def _sc_conv2(tab, edges, zeros4):
    return pl.kernel(
        _sc_conv2_body,
        out_type=jax.ShapeDtypeStruct((2, R, W2), jnp.float32),
        mesh=_mesh(),
        compiler_params=_SC_PARAMS,
        scratch_types=[
            pltpu.VMEM((4, 8, 128), jnp.int32),
            pltpu.VMEM((4, 8, 128), jnp.int32),
            pltpu.VMEM((2, 1024, W2), jnp.float32),
            pltpu.VMEM((WCH, W2), jnp.float32),
            pltpu.VMEM_SHARED((R, W2), jnp.float32),
            pltpu.SemaphoreType.DMA,
            pltpu.SemaphoreType.DMA,
            pltpu.SemaphoreType.DMA,
        ],
    )(tab, edges, zeros4)


def _encoder(x, p):
    h = jnp.maximum(jnp.dot(x, p[0], preferred_element_type=jnp.float32) + p[1], 0.0)
    for j in range(4):
        h = h + jnp.maximum(
            jnp.dot(h, p[2 + 2 * j], preferred_element_type=jnp.float32)
            + p[3 + 2 * j],
            0.0,
        )
    return jnp.dot(h, p[10], preferred_element_type=jnp.float32) + p[11]


def _dense1_body(*refs):
    xu_ref, xi_ref, deg_ref = refs[0], refs[1], refs[2]
    pu = [r[...] for r in refs[3:15]]
    pi = [r[...] for r in refs[15:27]]
    w1u, w1i = refs[27][...], refs[28][...]
    ou_ref, oi_ref = refs[29], refs[30]

    dd = deg_ref[...]
    hu = _encoder(xu_ref[...], pu)
    hi = _encoder(xi_ref[...], pi)
    su = lax.rsqrt(jnp.maximum(dd[0, :, 0:1], 1.0))
    si = lax.rsqrt(jnp.maximum(dd[2, :, 0:1], 1.0))
    yu = jnp.dot(hu, w1u, preferred_element_type=jnp.float32) * su
    yi = jnp.dot(hi, w1i, preferred_element_type=jnp.float32) * si
    for q in range(4):
        ou_ref[q] = yu[:, q * QW:(q + 1) * QW]
        oi_ref[q] = yi[:, q * QW:(q + 1) * QW]


def _full_spec(a):
    nd = a.ndim
    return pl.BlockSpec(a.shape, lambda i, _nd=nd: (0,) * _nd)


def _tc_dense1(x_user, x_item, deg, pu, pi, w1u, w1i):
    in_specs = [
        pl.BlockSpec((BLK, D_IN), lambda i: (i, 0)),
        pl.BlockSpec((BLK, D_IN), lambda i: (i, 0)),
        pl.BlockSpec((4, BLK, W2), lambda i: (0, i, 0)),
    ]
    args = [x_user, x_item, deg]
    for a in list(pu) + list(pi) + [w1u, w1i]:
        in_specs.append(_full_spec(a))
        args.append(a)
    out_shape = [jax.ShapeDtypeStruct((4, R, QW), jnp.float32)] * 2
    out_specs = [pl.BlockSpec((4, BLK, QW), lambda i: (0, i, 0))] * 2
    return pl.pallas_call(
        _dense1_body,
        grid=(NBLK,),
        in_specs=in_specs,
        out_specs=out_specs,
        out_shape=out_shape,
    )(*args)


def _prep2_body(m_ref, deg_ref, w2u_ref, w2i_ref, o_ref):
    mm = m_ref[...]
    dd = deg_ref[...]
    h_u1 = jnp.maximum(mm[1] * lax.rsqrt(jnp.maximum(dd[3, :, 0:1], 1.0)), 0.0)
    h_i1 = jnp.maximum(mm[0] * lax.rsqrt(jnp.maximum(dd[1, :, 0:1], 1.0)), 0.0)
    o_ref[0] = (
        jnp.dot(h_u1, w2u_ref[...], preferred_element_type=jnp.float32)
        * lax.rsqrt(jnp.maximum(dd[0, :, 0:1], 1.0))
    )
    o_ref[1] = (
        jnp.dot(h_i1, w2i_ref[...], preferred_element_type=jnp.float32)
        * lax.rsqrt(jnp.maximum(dd[2, :, 0:1], 1.0))
    )


def _tc_prep2(m64, deg, w2u, w2i):
    return pl.pallas_call(
        _prep2_body,
        grid=(NBLK,),
        in_specs=[
            pl.BlockSpec((2, BLK, D_H), lambda i: (0, i, 0)),
            pl.BlockSpec((4, BLK, W2), lambda i: (0, i, 0)),
            _full_spec(w2u),
            _full_spec(w2i),
        ],
        out_specs=pl.BlockSpec((2, BLK, W2), lambda i: (0, i, 0)),
        out_shape=jax.ShapeDtypeStruct((2, R, W2), jnp.float32),
    )(m64, deg, w2u, w2i)


def _final_body(o_ref, deg_ref, out_ref):
    oo = o_ref[...]
    dd = deg_ref[...]
    out_ref[0] = oo[0] * lax.rsqrt(jnp.maximum(dd[1, :, 0:1], 1.0))
    out_ref[1] = oo[1] * lax.rsqrt(jnp.maximum(dd[3, :, 0:1], 1.0))


def _tc_final(o_pre, deg):
    return pl.pallas_call(
        _final_body,
        grid=(NBLK,),
        in_specs=[
            pl.BlockSpec((2, BLK, W2), lambda i: (0, i, 0)),
            pl.BlockSpec((4, BLK, W2), lambda i: (0, i, 0)),
        ],
        out_specs=pl.BlockSpec((2, BLK, W2), lambda i: (0, i, 0)),
        out_shape=jax.ShapeDtypeStruct((2, R, W2), jnp.float32),
    )(o_pre, deg)


def _pad_edges(ei):
    pad = jnp.full((EP - E,), DUMMY, jnp.int32)
    return jnp.stack(
        [jnp.concatenate([ei[0], pad]), jnp.concatenate([ei[1], pad])]
    )


def kernel(x_user, x_item, ei_user_item, ei_item_user, enc_user_W_in, enc_user_b_in, enc_user_W0, enc_user_b0, enc_user_W1, enc_user_b1, enc_user_W2, enc_user_b2, enc_user_W3, enc_user_b3, enc_user_W_out, enc_user_b_out, enc_item_W_in, enc_item_b_in, enc_item_W0, enc_item_b0, enc_item_W1, enc_item_b1, enc_item_W2, enc_item_b2, enc_item_W3, enc_item_b3, enc_item_W_out, enc_item_b_out, gcn1_W_ui, gcn1_W_iu, gcn2_W_ui, gcn2_W_iu):
    edges = jnp.stack(
        [_pad_edges(ei_user_item), _pad_edges(ei_item_user)]
    ).reshape(2, 2, EROWS, 128)

    ones4 = jnp.ones((128, W2), jnp.float32)
    zeros4 = jnp.zeros((WCH, W2), jnp.float32)
    zeros16 = jnp.zeros((WCH, QW), jnp.float32)

    deg = _sc_degrees(edges, ones4, zeros4)

    pu = [enc_user_W_in, enc_user_b_in.reshape(1, D_H),
          enc_user_W0, enc_user_b0.reshape(1, D_H),
          enc_user_W1, enc_user_b1.reshape(1, D_H),
          enc_user_W2, enc_user_b2.reshape(1, D_H),
          enc_user_W3, enc_user_b3.reshape(1, D_H),
          enc_user_W_out, enc_user_b_out.reshape(1, D_H)]
    pi = [enc_item_W_in, enc_item_b_in.reshape(1, D_H),
          enc_item_W0, enc_item_b0.reshape(1, D_H),
          enc_item_W1, enc_item_b1.reshape(1, D_H),
          enc_item_W2, enc_item_b2.reshape(1, D_H),
          enc_item_W3, enc_item_b3.reshape(1, D_H),
          enc_item_W_out, enc_item_b_out.reshape(1, D_H)]

    xs1_u, xs1_i = _tc_dense1(x_user, x_item, deg, pu, pi, gcn1_W_ui, gcn1_W_iu)

    m64 = _sc_conv1(xs1_u, xs1_i, edges, zeros16)

    w2u = jnp.pad(gcn2_W_ui, ((0, 0), (0, W2 - D_OUT)))
    w2i = jnp.pad(gcn2_W_iu, ((0, 0), (0, W2 - D_OUT)))
    xs2 = _tc_prep2(m64, deg, w2u, w2i)

    o_pre = _sc_conv2(xs2, edges, zeros4)
    o = _tc_final(o_pre, deg)

    return jnp.concatenate([o[1, :N, :D_OUT], o[0, :N, :D_OUT]], axis=0)
